# baseline (speedup 1.0000x reference)
import functools

import jax
import jax.numpy as jnp
from jax import lax
from jax.experimental import pallas as pl
from jax.experimental.pallas import tpu as pltpu
from jax.experimental.pallas import tpu_sc as plsc

B, NATN, NARGS, NATNARG = 8, 8, 256, 2
H = 512
NKEYS = 8192

SPLITS = ((0, 4), (4, 4))
NW = 32
CH = 16


def _sc_gather_kernel(b0, nb, keyse_hbm, keyso_hbm, nmap_hbm, embed_hbm,
                      targs_hbm,
                      nm_v, keyse_v, keyso_v, idx_e, idx_o,
                      rowe0_v, rowe1_v, rowo0_v, rowo1_v, out0_v, out1_v,
                      gsem0, gsem1, wsem0, wsem1):
    ipw = nb * NATN * NARGS // NW
    nchunk = ipw // CH
    nc = 2
    wid = lax.axis_index("s") * nc + lax.axis_index("c")
    kbase = b0 * NATN * NARGS + wid * ipw
    ibase = wid * ipw

    pltpu.sync_copy(nmap_hbm, nm_v)
    pltpu.sync_copy(keyse_hbm.at[pl.ds(kbase, ipw)], keyse_v)
    pltpu.sync_copy(keyso_hbm.at[pl.ds(kbase, ipw)], keyso_v)

    def name_body(i, _):
        s = pl.ds(i * 16, 16)
        idx_e[s] = plsc.load_gather(nm_v, [keyse_v[s]])
        idx_o[s] = plsc.load_gather(nm_v, [keyso_v[s]])
        return 0

    lax.fori_loop(0, ipw // 16, name_body, 0)

    ebufs = (rowe0_v, rowe1_v)
    obufs = (rowo0_v, rowo1_v)
    outs = (out0_v, out1_v)
    gsems = (gsem0, gsem1)
    wsems = (wsem0, wsem1)

    def start_gathers(g, p):
        de = pltpu.async_copy(
            embed_hbm.at[idx_e.at[pl.ds(g * CH, CH)]], ebufs[p], gsems[p])
        do = pltpu.async_copy(
            embed_hbm.at[idx_o.at[pl.ds(g * CH, CH)]], obufs[p], gsems[p])
        return (de, do)

    gathers = [None, None]
    writes = [None, None]
    gathers[0] = start_gathers(0, 0)
    for g in range(nchunk):
        p = g % 2
        for d in gathers[p]:
            d.wait()
        if g + 1 < nchunk:
            gathers[1 - p] = start_gathers(g + 1, 1 - p)
        if writes[p] is not None:
            writes[p].wait()
        be, bo, out = ebufs[p], obufs[p], outs[p]

        def add_body(i, _):
            for hc in range(H // 16):
                s = pl.ds(hc * 16, 16)
                out[i, s] = be[i, s] + bo[i, s]
            return 0

        lax.fori_loop(0, CH, add_body, 0)
        w = pltpu.make_async_copy(
            out, targs_hbm.at[pl.ds(ibase + g * CH, CH)], wsems[p])
        w.start()
        writes[p] = w
    writes[0].wait()
    writes[1].wait()


def _sc_gather(keysE, keysO, nameMap, embed, b0, nb):
    ipw = nb * NATN * NARGS // NW
    mesh = plsc.VectorSubcoreMesh(core_axis_name="c", subcore_axis_name="s")
    fn = functools.partial(
        pl.kernel,
        mesh=mesh,
        out_type=jax.ShapeDtypeStruct((nb * NATN * NARGS, H), jnp.float32),
        scratch_types=[
            pltpu.VMEM((NKEYS,), jnp.int32),
            pltpu.VMEM((ipw,), jnp.int32),
            pltpu.VMEM((ipw,), jnp.int32),
            pltpu.VMEM((ipw,), jnp.int32),
            pltpu.VMEM((ipw,), jnp.int32),
            pltpu.VMEM((CH, H), jnp.float32),
            pltpu.VMEM((CH, H), jnp.float32),
            pltpu.VMEM((CH, H), jnp.float32),
            pltpu.VMEM((CH, H), jnp.float32),
            pltpu.VMEM((CH, H), jnp.float32),
            pltpu.VMEM((CH, H), jnp.float32),
            pltpu.SemaphoreType.DMA,
            pltpu.SemaphoreType.DMA,
            pltpu.SemaphoreType.DMA,
            pltpu.SemaphoreType.DMA,
        ],
        compiler_params=pltpu.CompilerParams(needs_layout_passes=False),
    )(functools.partial(_sc_gather_kernel, b0, nb))
    return fn(keysE, keysO, nameMap, embed)


def _tc_kernel(b0, lens_ref, targs_ref, wv_ref, bv_ref, stims_ref, wk_ref,
               bk_ref, x_ref, idx_ref, k_scr):
    b = pl.program_id(0)
    bg = b0 + b

    @pl.when(b == 0)
    def _():
        k_scr[...] = jnp.maximum(
            jax.lax.dot_general(stims_ref[...], wk_ref[...],
                                (((1,), (0,)), ((), ()))) + bk_ref[...], 0.0)

    v = jnp.maximum(
        jax.lax.dot_general(targs_ref[0], wv_ref[...],
                            (((1,), (0,)), ((), ()))) + bv_ref[...], 0.0)
    kb = k_scr[pl.ds(bg, 1), :]
    xcol = jax.lax.dot_general(v, kb, (((1,), (1,)), ((), ())),
                               precision=jax.lax.Precision.HIGHEST)
    xrow = jnp.swapaxes(xcol, 0, 1)
    x_ref[0] = xrow

    ids = lax.broadcasted_iota(jnp.int32, (1, NARGS), 1)
    out = jnp.zeros((1, 128), jnp.int32)
    lane = lax.broadcasted_iota(jnp.int32, (1, 128), 1)
    for a in range(NATN):
        xa = xrow[:, a * NARGS:(a + 1) * NARGS]
        masked = jnp.where(ids < lens_ref[bg, a], xa, -1e9)
        xmax = jnp.max(masked)
        amin = jnp.min(jnp.where(masked == xmax, ids, NARGS))
        out = jnp.where(lane == a, amin, out)
    idx_ref[0] = out


def _tc_stage(targs, Wv, bv, stims, Wk, bk, atnLens, b0, nb):
    x, idx = pl.pallas_call(
        functools.partial(_tc_kernel, b0),
        grid=(nb,),
        in_specs=[
            pl.BlockSpec(memory_space=pltpu.SMEM),
            pl.BlockSpec((1, NATN * NARGS, H), lambda i: (i, 0, 0)),
            pl.BlockSpec((H, H), lambda i: (0, 0)),
            pl.BlockSpec((1, H), lambda i: (0, 0)),
            pl.BlockSpec((B, H), lambda i: (0, 0)),
            pl.BlockSpec((H, H), lambda i: (0, 0)),
            pl.BlockSpec((1, H), lambda i: (0, 0)),
        ],
        out_specs=[
            pl.BlockSpec((1, 1, NATN * NARGS), lambda i: (i, 0, 0)),
            pl.BlockSpec((1, 1, 128), lambda i: (i, 0, 0)),
        ],
        out_shape=[
            jax.ShapeDtypeStruct((nb, 1, NATN * NARGS), jnp.float32),
            jax.ShapeDtypeStruct((nb, 1, 128), jnp.int32),
        ],
        scratch_shapes=[pltpu.VMEM((B, H), jnp.float32)],
    )(atnLens, targs.reshape(nb, NATN * NARGS, H), Wv, bv, stims, Wk, bk)
    return x, idx


def kernel(stims, atnTensor, atnLens, nameMap, embed, Wk, bk, Wv, bv):
    at = atnTensor.astype(jnp.int32)
    keysE = at[:, :, :, 0, 0].reshape(-1)
    keysO = at[:, :, :, 1, 0].reshape(-1)
    nmap = nameMap.astype(jnp.int32)
    bv2 = bv.reshape(1, H)
    bk2 = bk.reshape(1, H)
    xs, idxs = [], []
    for b0, nb in SPLITS:
        targs = _sc_gather(keysE, keysO, nmap, embed, b0, nb)
        x, idx = _tc_stage(targs, Wv, bv2, stims, Wk, bk2, atnLens, b0, nb)
        xs.append(x)
        idxs.append(idx)
    x = jnp.concatenate(xs, axis=0)
    idx = jnp.concatenate(idxs, axis=0)
    xIdx = idx[:, 0, :NATN].astype(jnp.int32)
    return (x.reshape(B, NATN, NARGS), xIdx)


# speedup vs baseline: 1.1455x; 1.1455x over previous
import functools

import jax
import jax.numpy as jnp
from jax import lax
from jax.experimental import pallas as pl
from jax.experimental.pallas import tpu as pltpu
from jax.experimental.pallas import tpu_sc as plsc

B, NATN, NARGS, NATNARG = 8, 8, 256, 2
H = 512
NKEYS = 8192

SPLITS = ((0, 4), (4, 4))
NW = 32
CH = 32


def _sc_gather_kernel(b0, nb, keyse_hbm, keyso_hbm, nmap_hbm, embed_hbm,
                      targs_hbm,
                      nm_v, keyse_v, keyso_v, idx_e, idx_o,
                      rowe0_v, rowe1_v, rowo0_v, rowo1_v, out0_v, out1_v,
                      gsem0, gsem1, wsem0, wsem1):
    ipw = nb * NATN * NARGS // NW
    nchunk = ipw // CH
    nc = 2
    wid = lax.axis_index("s") * nc + lax.axis_index("c")
    kbase = b0 * NATN * NARGS + wid * ipw
    ibase = wid * ipw

    pltpu.sync_copy(nmap_hbm, nm_v)
    pltpu.sync_copy(keyse_hbm.at[pl.ds(kbase, ipw)], keyse_v)
    pltpu.sync_copy(keyso_hbm.at[pl.ds(kbase, ipw)], keyso_v)

    def name_body(i, _):
        s = pl.ds(i * 16, 16)
        idx_e[s] = plsc.load_gather(nm_v, [keyse_v[s]])
        idx_o[s] = plsc.load_gather(nm_v, [keyso_v[s]])
        return 0

    lax.fori_loop(0, ipw // 16, name_body, 0)

    ebufs = (rowe0_v, rowe1_v)
    obufs = (rowo0_v, rowo1_v)
    outs = (out0_v, out1_v)
    gsems = (gsem0, gsem1)
    wsems = (wsem0, wsem1)

    def start_gathers(g, p):
        de = pltpu.async_copy(
            embed_hbm.at[idx_e.at[pl.ds(g * CH, CH)]], ebufs[p], gsems[p])
        do = pltpu.async_copy(
            embed_hbm.at[idx_o.at[pl.ds(g * CH, CH)]], obufs[p], gsems[p])
        return (de, do)

    gathers = [None, None]
    writes = [None, None]
    gathers[0] = start_gathers(0, 0)
    for g in range(nchunk):
        p = g % 2
        for d in gathers[p]:
            d.wait()
        if g + 1 < nchunk:
            gathers[1 - p] = start_gathers(g + 1, 1 - p)
        if writes[p] is not None:
            writes[p].wait()
        be, bo, out = ebufs[p], obufs[p], outs[p]

        def add_body(i, _):
            for hc in range(H // 16):
                s = pl.ds(hc * 16, 16)
                out[i, s] = be[i, s] + bo[i, s]
            return 0

        lax.fori_loop(0, CH, add_body, 0)
        w = pltpu.make_async_copy(
            out, targs_hbm.at[pl.ds(ibase + g * CH, CH)], wsems[p])
        w.start()
        writes[p] = w
    writes[0].wait()
    writes[1].wait()


def _sc_gather(keysE, keysO, nameMap, embed, b0, nb):
    ipw = nb * NATN * NARGS // NW
    mesh = plsc.VectorSubcoreMesh(core_axis_name="c", subcore_axis_name="s")
    fn = functools.partial(
        pl.kernel,
        mesh=mesh,
        out_type=jax.ShapeDtypeStruct((nb * NATN * NARGS, H), jnp.float32),
        scratch_types=[
            pltpu.VMEM((NKEYS,), jnp.int32),
            pltpu.VMEM((ipw,), jnp.int32),
            pltpu.VMEM((ipw,), jnp.int32),
            pltpu.VMEM((ipw,), jnp.int32),
            pltpu.VMEM((ipw,), jnp.int32),
            pltpu.VMEM((CH, H), jnp.float32),
            pltpu.VMEM((CH, H), jnp.float32),
            pltpu.VMEM((CH, H), jnp.float32),
            pltpu.VMEM((CH, H), jnp.float32),
            pltpu.VMEM((CH, H), jnp.float32),
            pltpu.VMEM((CH, H), jnp.float32),
            pltpu.SemaphoreType.DMA,
            pltpu.SemaphoreType.DMA,
            pltpu.SemaphoreType.DMA,
            pltpu.SemaphoreType.DMA,
        ],
        compiler_params=pltpu.CompilerParams(needs_layout_passes=False),
    )(functools.partial(_sc_gather_kernel, b0, nb))
    return fn(keysE, keysO, nameMap, embed)


def _tc_kernel(b0, lens_ref, targs_ref, wv_ref, bv_ref, stims_ref, wk_ref,
               bk_ref, x_ref, idx_ref, k_scr):
    b = pl.program_id(0)
    bg = b0 + b

    @pl.when(b == 0)
    def _():
        k_scr[...] = jnp.maximum(
            jax.lax.dot_general(stims_ref[...], wk_ref[...],
                                (((1,), (0,)), ((), ()))) + bk_ref[...], 0.0)

    v = jnp.maximum(
        jax.lax.dot_general(targs_ref[0], wv_ref[...],
                            (((1,), (0,)), ((), ()))) + bv_ref[...], 0.0)
    kb = k_scr[pl.ds(bg, 1), :]
    xcol = jax.lax.dot_general(v, kb, (((1,), (1,)), ((), ())),
                               precision=jax.lax.Precision.HIGHEST)
    xrow = jnp.swapaxes(xcol, 0, 1)
    x_ref[0] = xrow

    ids = lax.broadcasted_iota(jnp.int32, (1, NARGS), 1)
    out = jnp.zeros((1, 128), jnp.int32)
    lane = lax.broadcasted_iota(jnp.int32, (1, 128), 1)
    for a in range(NATN):
        xa = xrow[:, a * NARGS:(a + 1) * NARGS]
        masked = jnp.where(ids < lens_ref[bg, a], xa, -1e9)
        xmax = jnp.max(masked)
        amin = jnp.min(jnp.where(masked == xmax, ids, NARGS))
        out = jnp.where(lane == a, amin, out)
    idx_ref[0] = out


def _tc_stage(targs, Wv, bv, stims, Wk, bk, atnLens, b0, nb):
    x, idx = pl.pallas_call(
        functools.partial(_tc_kernel, b0),
        grid=(nb,),
        in_specs=[
            pl.BlockSpec(memory_space=pltpu.SMEM),
            pl.BlockSpec((1, NATN * NARGS, H), lambda i: (i, 0, 0)),
            pl.BlockSpec((H, H), lambda i: (0, 0)),
            pl.BlockSpec((1, H), lambda i: (0, 0)),
            pl.BlockSpec((B, H), lambda i: (0, 0)),
            pl.BlockSpec((H, H), lambda i: (0, 0)),
            pl.BlockSpec((1, H), lambda i: (0, 0)),
        ],
        out_specs=[
            pl.BlockSpec((1, 1, NATN * NARGS), lambda i: (i, 0, 0)),
            pl.BlockSpec((1, 1, 128), lambda i: (i, 0, 0)),
        ],
        out_shape=[
            jax.ShapeDtypeStruct((nb, 1, NATN * NARGS), jnp.float32),
            jax.ShapeDtypeStruct((nb, 1, 128), jnp.int32),
        ],
        scratch_shapes=[pltpu.VMEM((B, H), jnp.float32)],
    )(atnLens, targs.reshape(nb, NATN * NARGS, H), Wv, bv, stims, Wk, bk)
    return x, idx


def kernel(stims, atnTensor, atnLens, nameMap, embed, Wk, bk, Wv, bv):
    at = atnTensor.astype(jnp.int32)
    keysE = at[:, :, :, 0, 0].reshape(-1)
    keysO = at[:, :, :, 1, 0].reshape(-1)
    nmap = nameMap.astype(jnp.int32)
    bv2 = bv.reshape(1, H)
    bk2 = bk.reshape(1, H)
    xs, idxs = [], []
    for b0, nb in SPLITS:
        targs = _sc_gather(keysE, keysO, nmap, embed, b0, nb)
        x, idx = _tc_stage(targs, Wv, bv2, stims, Wk, bk2, atnLens, b0, nb)
        xs.append(x)
        idxs.append(idx)
    x = jnp.concatenate(xs, axis=0)
    idx = jnp.concatenate(idxs, axis=0)
    xIdx = idx[:, 0, :NATN].astype(jnp.int32)
    return (x.reshape(B, NATN, NARGS), xIdx)
